# baseline (speedup 1.0000x reference)
import functools

import jax
import jax.numpy as jnp
import numpy as np
from jax import lax
from jax.experimental import pallas as pl
from jax.experimental.pallas import tpu as pltpu
from jax.experimental.pallas import tpu_sc as plsc

_FIELD_DIMS = [100000] * 26
_F = len(_FIELD_DIMS)
_D = 16
_BATCH = 4096

_NC = 2
_NS = 16
_NW = _NC * _NS
_BW = _BATCH // _NW
_IW = _BW * _F
_G16 = _BW // 16

_OFFSETS = np.concatenate(([0], np.cumsum(_FIELD_DIMS)[:-1])).astype(np.int32)
_OFF_REP = np.tile(_OFFSETS, _BW)


def _fm_body(x_hbm, off_hbm, table_hbm, out_hbm,
             idx_v, off_v, rows_v, tbuf_v, out_v, sem):
    wid = lax.axis_index("s") * _NC + lax.axis_index("c")
    ibase = wid * _IW
    obase = wid * _BW

    pltpu.sync_copy(x_hbm.at[pl.ds(ibase, _IW)], idx_v)
    pltpu.sync_copy(off_hbm, off_v)

    def add_off(i, c):
        sl = pl.ds(i * 16, 16)
        idx_v[sl] = idx_v[sl] + off_v[sl]
        return c
    lax.fori_loop(0, _IW // 16, add_off, 0)

    pltpu.async_copy(table_hbm.at[idx_v], rows_v, sem).wait()

    zero = jnp.zeros((_D,), jnp.float32)

    def elem_body(b, c):
        rb = b * _F
        s = zero
        q = zero
        for f in range(_F):
            r = rows_v[rb + f, :]
            s = s + r
            q = q + r * r
        tbuf_v[pl.ds(b * _D, _D)] = s * s - q
        return c
    lax.fori_loop(0, _BW, elem_body, 0)

    lanes = lax.iota(jnp.int32, 16)

    def group_body(g, c):
        flat_base = (g * 16 + lanes) * _D
        acc = zero
        for d in range(_D):
            col = plsc.load_gather(tbuf_v, [flat_base + d])
            acc = acc + col
        z = 0.5 * acc
        out_v[pl.ds(g * 16, 16)] = 1.0 / (1.0 + jnp.exp(-z))
        return c
    lax.fori_loop(0, _G16, group_body, 0)

    pltpu.sync_copy(out_v, out_hbm.at[pl.ds(obase, _BW)])


_fm_kernel = functools.partial(
    pl.kernel,
    out_type=jax.ShapeDtypeStruct((_BATCH,), jnp.float32),
    mesh=plsc.VectorSubcoreMesh(core_axis_name="c", subcore_axis_name="s"),
    compiler_params=pltpu.CompilerParams(
        needs_layout_passes=False, use_tc_tiling_on_sc=False),
    scratch_types=[
        pltpu.VMEM((_IW,), jnp.int32),
        pltpu.VMEM((_IW,), jnp.int32),
        pltpu.VMEM((_IW, _D), jnp.float32),
        pltpu.VMEM((_BW * _D,), jnp.float32),
        pltpu.VMEM((_BW,), jnp.float32),
        pltpu.SemaphoreType.DMA,
    ],
)(_fm_body)


def kernel(x, table):
    x_flat = x.reshape(-1).astype(jnp.int32)
    off_rep = jnp.asarray(_OFF_REP)
    return _fm_kernel(x_flat, off_rep, table)
